# Initial kernel scaffold; baseline (speedup 1.0000x reference)
import jax
import jax.numpy as jnp
from jax.experimental import pallas as pl


def kernel(embeddings, positions, Wq, Wk, Wv, W0):
    raise NotImplementedError("write your pallas kernel here")


# speedup vs baseline: 8.0247x; 8.0247x over previous
import jax
import jax.numpy as jnp
from jax import lax
from jax.experimental import pallas as pl
from jax.experimental.pallas import tpu as pltpu

_H = 8
_DH = 16
_TQ = 128
_TK = 384
_BAND = 64


def _attn_body(mc_ref, xs_ref, pos_ref, wq_ref, wk_ref, wv_ref, w0_ref, ys_ref):
    S = xs_ref.shape[1]
    D = xs_ref.shape[2]
    t0 = pl.multiple_of(pl.program_id(1) * _TQ, _TQ)
    w0 = pl.multiple_of(jnp.clip(t0 - (_TK - _TQ) // 2, 0, S - _TK), _TQ)

    qx = xs_ref[0, pl.ds(t0, _TQ), :]
    xw = xs_ref[0, pl.ds(w0, _TK), :]
    q = jnp.dot(qx, wq_ref[...], preferred_element_type=jnp.float32)
    k = jnp.dot(xw, wk_ref[...], preferred_element_type=jnp.float32)
    v = jnp.dot(xw, wv_ref[...], preferred_element_type=jnp.float32)

    pq = pos_ref[0, :, pl.ds(t0, _TQ)]
    pk = pos_ref[0, :, pl.ds(w0, _TK)]
    ii = lax.broadcasted_iota(jnp.int32, (_TQ, _TQ), 0)
    jj = lax.broadcasted_iota(jnp.int32, (_TQ, _TQ), 1)
    eye = (ii == jj).astype(jnp.float32)
    pqc = lax.dot_general(eye, pq, (((1,), (1,)), ((), ())),
                          preferred_element_type=jnp.float32)

    eq = pqc == pk
    cnt = jnp.sum(eq.astype(jnp.float32), axis=1, keepdims=True)
    mc = mc_ref[0, 0].astype(jnp.float32)

    lane = lax.broadcasted_iota(jnp.int32, (_TQ, D), 1) // _DH
    ctx = jnp.zeros((_TQ, D), jnp.float32)
    for h in range(_H):
        qm = jnp.where(lane == h, q, 0.0)
        s = lax.dot_general(qm, k, (((1,), (1,)), ((), ())),
                            preferred_element_type=jnp.float32) * 0.25
        s = jnp.where(eq, s, -1e30)
        m = jnp.maximum(jnp.max(s, axis=1, keepdims=True), 0.0)
        e = jnp.where(eq, jnp.exp(s - m), 0.0)
        denom = jnp.sum(e, axis=1, keepdims=True) + (mc - cnt) * jnp.exp(-m)
        attn = e / denom
        ph = jnp.dot(attn, v, preferred_element_type=jnp.float32)
        ctx = ctx + jnp.where(lane == h, ph, 0.0)
    ys_ref[0] = jnp.dot(ctx, w0_ref[...], preferred_element_type=jnp.float32)


def _banded_attention(mc, xs, spf, Wq, Wk, Wv, W0):
    B, S, D = xs.shape
    grid = (B, S // _TQ)
    return pl.pallas_call(
        _attn_body,
        grid=grid,
        in_specs=[
            pl.BlockSpec(memory_space=pltpu.SMEM),
            pl.BlockSpec((1, S, D), lambda b, t: (b, 0, 0)),
            pl.BlockSpec((1, 1, S), lambda b, t: (b, 0, 0)),
            pl.BlockSpec((D, D), lambda b, t: (0, 0)),
            pl.BlockSpec((D, D), lambda b, t: (0, 0)),
            pl.BlockSpec((D, D), lambda b, t: (0, 0)),
            pl.BlockSpec((D, D), lambda b, t: (0, 0)),
        ],
        out_specs=pl.BlockSpec((1, _TQ, D), lambda b, t: (b, t, 0)),
        out_shape=jax.ShapeDtypeStruct((B, S, D), jnp.float32),
    )(mc, xs, spf, Wq, Wk, Wv, W0)


def kernel(embeddings, positions, Wq, Wk, Wv, W0):
    B, S, D = embeddings.shape
    perm = jnp.argsort(positions, axis=1).astype(jnp.int32)
    sp = jnp.take_along_axis(positions, perm, axis=1)
    xs = jnp.take_along_axis(embeddings, perm[..., None], axis=1)

    idx = jnp.arange(S, dtype=jnp.int32)[None, :]
    start = jnp.concatenate(
        [jnp.ones((B, 1), bool), sp[:, 1:] != sp[:, :-1]], axis=1)
    last_start = lax.cummax(jnp.where(start, idx, 0), axis=1)
    mc = (jnp.max(idx - last_start) + 1).reshape(1, 1)

    spf = sp.astype(jnp.float32).reshape(B, 1, S)
    ys = _banded_attention(mc, xs, spf, Wq, Wk, Wv, W0)

    inv = jnp.argsort(perm, axis=1)
    return jnp.take_along_axis(ys, inv[..., None], axis=1)


# speedup vs baseline: 10.8227x; 1.3487x over previous
import jax
import jax.numpy as jnp
from jax import lax
from jax.experimental import pallas as pl
from jax.experimental.pallas import tpu as pltpu

_H = 8
_DH = 16
_TQ = 128
_TK = 384
_BAND = 64


def _attn_body(mc_ref, xs_ref, pos_ref, wq_ref, wk_ref, wv_ref, w0_ref, ys_ref):
    S = xs_ref.shape[1]
    D = xs_ref.shape[2]
    t0 = pl.multiple_of(pl.program_id(1) * _TQ, _TQ)
    w0 = pl.multiple_of(jnp.clip(t0 - (_TK - _TQ) // 2, 0, S - _TK), _TQ)

    qx = xs_ref[0, pl.ds(t0, _TQ), :]
    xw = xs_ref[0, pl.ds(w0, _TK), :]
    q = jnp.dot(qx, wq_ref[...], preferred_element_type=jnp.float32)
    k = jnp.dot(xw, wk_ref[...], preferred_element_type=jnp.float32)
    v = jnp.dot(xw, wv_ref[...], preferred_element_type=jnp.float32)

    pq = pos_ref[0, :, pl.ds(t0, _TQ)]
    pk = pos_ref[0, :, pl.ds(w0, _TK)]
    ii = lax.broadcasted_iota(jnp.int32, (_TQ, _TQ), 0)
    jj = lax.broadcasted_iota(jnp.int32, (_TQ, _TQ), 1)
    eye = (ii == jj).astype(jnp.float32)
    pqc = lax.dot_general(eye, pq, (((1,), (1,)), ((), ())),
                          preferred_element_type=jnp.float32)

    eq = pqc == pk
    bias = jnp.where(eq, 0.0, -1e30)
    cnt = jnp.sum(eq.astype(jnp.float32), axis=1, keepdims=True)
    mc = mc_ref[0, 0].astype(jnp.float32)

    lane = lax.broadcasted_iota(jnp.int32, (_TQ, D), 1) // _DH
    qm = jnp.concatenate([jnp.where(lane == h, q, 0.0) for h in range(_H)],
                         axis=0)
    s = lax.dot_general(qm, k, (((1,), (1,)), ((), ())),
                        preferred_element_type=jnp.float32) * 0.25
    s = s + jnp.concatenate([bias] * _H, axis=0)
    m = jnp.maximum(jnp.max(s, axis=1, keepdims=True), 0.0)
    e = jnp.exp(s - m)
    cnt_t = jnp.concatenate([cnt] * _H, axis=0)
    denom = jnp.sum(e, axis=1, keepdims=True) + (mc - cnt_t) * jnp.exp(-m)
    rd = 1.0 / denom
    cs = jnp.dot(e, v, preferred_element_type=jnp.float32)
    ctx = jnp.zeros((_TQ, D), jnp.float32)
    for h in range(_H):
        ph = cs[h * _TQ:(h + 1) * _TQ] * rd[h * _TQ:(h + 1) * _TQ]
        ctx = ctx + jnp.where(lane == h, ph, 0.0)
    ys_ref[0] = jnp.dot(ctx, w0_ref[...], preferred_element_type=jnp.float32)


def _banded_attention(mc, xs, spf, Wq, Wk, Wv, W0):
    B, S, D = xs.shape
    grid = (B, S // _TQ)
    return pl.pallas_call(
        _attn_body,
        grid=grid,
        in_specs=[
            pl.BlockSpec(memory_space=pltpu.SMEM),
            pl.BlockSpec((1, S, D), lambda b, t: (b, 0, 0)),
            pl.BlockSpec((1, 1, S), lambda b, t: (b, 0, 0)),
            pl.BlockSpec((D, D), lambda b, t: (0, 0)),
            pl.BlockSpec((D, D), lambda b, t: (0, 0)),
            pl.BlockSpec((D, D), lambda b, t: (0, 0)),
            pl.BlockSpec((D, D), lambda b, t: (0, 0)),
        ],
        out_specs=pl.BlockSpec((1, _TQ, D), lambda b, t: (b, t, 0)),
        out_shape=jax.ShapeDtypeStruct((B, S, D), jnp.float32),
    )(mc, xs, spf, Wq, Wk, Wv, W0)


def kernel(embeddings, positions, Wq, Wk, Wv, W0):
    B, S, D = embeddings.shape
    perm = jnp.argsort(positions, axis=1).astype(jnp.int32)
    sp = jnp.take_along_axis(positions, perm, axis=1)
    xs = jnp.take_along_axis(embeddings, perm[..., None], axis=1)

    idx = jnp.arange(S, dtype=jnp.int32)[None, :]
    start = jnp.concatenate(
        [jnp.ones((B, 1), bool), sp[:, 1:] != sp[:, :-1]], axis=1)
    last_start = lax.cummax(jnp.where(start, idx, 0), axis=1)
    mc = (jnp.max(idx - last_start) + 1).reshape(1, 1)

    spf = sp.astype(jnp.float32).reshape(B, 1, S)
    ys = _banded_attention(mc, xs, spf, Wq, Wk, Wv, W0)

    inv = jnp.argsort(perm, axis=1)
    return jnp.take_along_axis(ys, inv[..., None], axis=1)


# speedup vs baseline: 10.8479x; 1.0023x over previous
import functools

import jax
import jax.numpy as jnp
from jax import lax
from jax.experimental import pallas as pl
from jax.experimental.pallas import tpu as pltpu
from jax.experimental.pallas import tpu_sc as plsc

_H = 8
_DH = 16
_TQ = 128
_TK = 384
_BAND = 64


def _attn_body(mc_ref, xs_ref, pos_ref, wq_ref, wk_ref, wv_ref, w0_ref, ys_ref):
    S = xs_ref.shape[1]
    D = xs_ref.shape[2]
    t0 = pl.multiple_of(pl.program_id(1) * _TQ, _TQ)
    w0 = pl.multiple_of(jnp.clip(t0 - (_TK - _TQ) // 2, 0, S - _TK), _TQ)

    qx = xs_ref[0, pl.ds(t0, _TQ), :]
    xw = xs_ref[0, pl.ds(w0, _TK), :]
    q = jnp.dot(qx, wq_ref[...], preferred_element_type=jnp.float32)
    k = jnp.dot(xw, wk_ref[...], preferred_element_type=jnp.float32)
    v = jnp.dot(xw, wv_ref[...], preferred_element_type=jnp.float32)

    pq = pos_ref[0, :, pl.ds(t0, _TQ)]
    pk = pos_ref[0, :, pl.ds(w0, _TK)]
    ii = lax.broadcasted_iota(jnp.int32, (_TQ, _TQ), 0)
    jj = lax.broadcasted_iota(jnp.int32, (_TQ, _TQ), 1)
    eye = (ii == jj).astype(jnp.float32)
    pqc = lax.dot_general(eye, pq, (((1,), (1,)), ((), ())),
                          preferred_element_type=jnp.float32)

    eq = pqc == pk
    bias = jnp.where(eq, 0.0, -1e30)
    cnt = jnp.sum(eq.astype(jnp.float32), axis=1, keepdims=True)
    mc = mc_ref[0, 0].astype(jnp.float32)

    lane = lax.broadcasted_iota(jnp.int32, (_TQ, D), 1) // _DH
    qm = jnp.concatenate([jnp.where(lane == h, q, 0.0) for h in range(_H)],
                         axis=0)
    s = lax.dot_general(qm, k, (((1,), (1,)), ((), ())),
                        preferred_element_type=jnp.float32) * 0.25
    s = s + jnp.concatenate([bias] * _H, axis=0)
    m = jnp.maximum(jnp.max(s, axis=1, keepdims=True), 0.0)
    e = jnp.exp(s - m)
    cnt_t = jnp.concatenate([cnt] * _H, axis=0)
    denom = jnp.sum(e, axis=1, keepdims=True) + (mc - cnt_t) * jnp.exp(-m)
    rd = 1.0 / denom
    cs = jnp.dot(e, v, preferred_element_type=jnp.float32)
    ctx = jnp.zeros((_TQ, D), jnp.float32)
    for h in range(_H):
        ph = cs[h * _TQ:(h + 1) * _TQ] * rd[h * _TQ:(h + 1) * _TQ]
        ctx = ctx + jnp.where(lane == h, ph, 0.0)
    ys_ref[0] = jnp.dot(ctx, w0_ref[...], preferred_element_type=jnp.float32)


def _banded_attention(mc, xs, spf, Wq, Wk, Wv, W0):
    B, S, D = xs.shape
    grid = (B, S // _TQ)
    return pl.pallas_call(
        _attn_body,
        grid=grid,
        in_specs=[
            pl.BlockSpec(memory_space=pltpu.SMEM),
            pl.BlockSpec((1, S, D), lambda b, t: (b, 0, 0)),
            pl.BlockSpec((1, 1, S), lambda b, t: (b, 0, 0)),
            pl.BlockSpec((D, D), lambda b, t: (0, 0)),
            pl.BlockSpec((D, D), lambda b, t: (0, 0)),
            pl.BlockSpec((D, D), lambda b, t: (0, 0)),
            pl.BlockSpec((D, D), lambda b, t: (0, 0)),
        ],
        out_specs=pl.BlockSpec((1, _TQ, D), lambda b, t: (b, t, 0)),
        out_shape=jax.ShapeDtypeStruct((B, S, D), jnp.float32),
    )(mc, xs, spf, Wq, Wk, Wv, W0)


def _sc_gather(table, idx):
    N, D = table.shape
    NW = 32
    CH = 128
    per_w = N // NW
    nch = per_w // CH
    idx3 = idx.reshape(NW, nch, CH)
    mesh = plsc.VectorSubcoreMesh(core_axis_name="c", subcore_axis_name="s")

    @functools.partial(
        pl.kernel, mesh=mesh,
        out_type=jax.ShapeDtypeStruct((N, D), jnp.float32),
        scratch_types=[
            pltpu.VMEM((nch, CH), jnp.int32),
            pltpu.VMEM((nch, CH, D), jnp.float32),
            pltpu.SemaphoreType.DMA,
        ],
    )
    def k(table_hbm, idx_hbm, out_hbm, idx_v, rows_v, sem):
        wid = lax.axis_index("s") * 2 + lax.axis_index("c")
        base = wid * per_w
        pltpu.sync_copy(idx_hbm.at[wid], idx_v)
        cps = [pltpu.async_copy(table_hbm.at[idx_v.at[j]], rows_v.at[j], sem)
               for j in range(nch)]
        for j in range(nch):
            cps[j].wait()
            pltpu.sync_copy(rows_v.at[j], out_hbm.at[pl.ds(base + j * CH, CH)])

    return k(table, idx3)


_TA = 256


def _maxgrp_body(pos_ref, mc_ref):
    S = pos_ref.shape[2]
    b = pl.program_id(0)
    t = pl.program_id(1)
    i0 = pl.multiple_of(t * _TA, _TA)
    a0 = pl.multiple_of(jnp.clip(i0 - 128, 0, S - _TK), 128)
    pq = pos_ref[0, :, pl.ds(i0, _TA)]
    pw = pos_ref[0, :, pl.ds(a0, _TK)]
    ii = lax.broadcasted_iota(jnp.int32, (_TA, _TA), 0)
    jj = lax.broadcasted_iota(jnp.int32, (_TA, _TA), 1)
    eye = (ii == jj).astype(jnp.float32)
    pqc = lax.dot_general(eye, pq, (((1,), (1,)), ((), ())),
                          preferred_element_type=jnp.float32)
    cnt = jnp.sum((pqc == pw).astype(jnp.int32), axis=1)
    mo = jnp.max(cnt)

    @pl.when((b == 0) & (t == 0))
    def _init():
        mc_ref[0, 0] = mo

    @pl.when(jnp.logical_not((b == 0) & (t == 0)))
    def _acc():
        mc_ref[0, 0] = jnp.maximum(mc_ref[0, 0], mo)


def _max_group(spf):
    B = spf.shape[0]
    S = spf.shape[2]
    return pl.pallas_call(
        _maxgrp_body,
        grid=(B, S // _TA),
        in_specs=[pl.BlockSpec((1, 1, S), lambda b, t: (b, 0, 0))],
        out_specs=pl.BlockSpec(memory_space=pltpu.SMEM),
        out_shape=jax.ShapeDtypeStruct((1, 1), jnp.int32),
    )(spf)


def kernel(embeddings, positions, Wq, Wk, Wv, W0):
    B, S, D = embeddings.shape
    perm = jnp.argsort(positions, axis=1).astype(jnp.int32)
    inv = jnp.argsort(perm, axis=1).astype(jnp.int32)
    sp = jnp.take_along_axis(positions, perm, axis=1)
    spf = sp.astype(jnp.float32).reshape(B, 1, S)

    row = (jnp.arange(B, dtype=jnp.int32) * S)[:, None]
    xs = _sc_gather(embeddings.reshape(B * S, D),
                    (perm + row).reshape(-1)).reshape(B, S, D)
    mc = _max_group(spf)
    ys = _banded_attention(mc, xs, spf, Wq, Wk, Wv, W0)
    return _sc_gather(ys.reshape(B * S, D),
                      (inv + row).reshape(-1)).reshape(B, S, D)


# speedup vs baseline: 12.8748x; 1.1868x over previous
import functools

import jax
import jax.numpy as jnp
from jax import lax
from jax.experimental import pallas as pl
from jax.experimental.pallas import tpu as pltpu
from jax.experimental.pallas import tpu_sc as plsc

_H = 8
_DH = 16
_TQ = 128
_TK = 256
_TW = 384
_BAND = 64


_NC = 2


def _attn_chunk(t0, mc_ref, xs_ref, pos_ref, wq_ref, wk_ref, wv_ref):
    S = xs_ref.shape[1]
    D = xs_ref.shape[2]
    w0 = pl.multiple_of(jnp.clip(t0 - (_TK - _TQ) // 2, 0, S - _TK), 64)

    qx = xs_ref[0, pl.ds(t0, _TQ), :]
    xw = xs_ref[0, pl.ds(w0, _TK), :]
    q = jnp.dot(qx, wq_ref[...], preferred_element_type=jnp.float32)
    k = jnp.dot(xw, wk_ref[...], preferred_element_type=jnp.float32)
    v = jnp.dot(xw, wv_ref[...], preferred_element_type=jnp.float32)

    pqc = pos_ref[0, pl.ds(t0, _TQ), :]
    pkc = pos_ref[0, pl.ds(w0, _TK), :]
    ii = lax.broadcasted_iota(jnp.int32, (_TK, _TK), 0)
    jj = lax.broadcasted_iota(jnp.int32, (_TK, _TK), 1)
    eye = (ii == jj).astype(jnp.float32)
    pk = lax.dot_general(pkc, eye, (((0,), (0,)), ((), ())),
                         preferred_element_type=jnp.float32)

    eq = pqc == pk
    bias = jnp.where(eq, 0.0, -1e30)
    cnt = jnp.sum(eq.astype(jnp.float32), axis=1, keepdims=True)
    mc = mc_ref[0, 0].astype(jnp.float32)

    lane = lax.broadcasted_iota(jnp.int32, (_TQ, D), 1) // _DH
    qm = jnp.concatenate([jnp.where(lane == h, q, 0.0) for h in range(_H)],
                         axis=0)
    s = lax.dot_general(qm, k, (((1,), (1,)), ((), ())),
                        preferred_element_type=jnp.float32)
    e = jnp.exp(s + jnp.concatenate([bias] * _H, axis=0))
    cnt_t = jnp.concatenate([cnt] * _H, axis=0)
    denom = jnp.sum(e, axis=1, keepdims=True) + (mc - cnt_t)
    rd = 1.0 / denom
    cs = jnp.dot(e, v, preferred_element_type=jnp.float32)
    ctx = jnp.zeros((_TQ, D), jnp.float32)
    for h in range(_H):
        ph = cs[h * _TQ:(h + 1) * _TQ] * rd[h * _TQ:(h + 1) * _TQ]
        ctx = ctx + jnp.where(lane == h, ph, 0.0)
    return ctx


def _attn_body(mc_ref, xs_ref, pos_ref, wq_ref, wk_ref, wv_ref, w0_ref, ys_ref):
    base = pl.multiple_of(pl.program_id(1) * (_NC * _TQ), _NC * _TQ)
    ctxs = [_attn_chunk(pl.multiple_of(base + c * _TQ, _TQ), mc_ref, xs_ref,
                        pos_ref, wq_ref, wk_ref, wv_ref)
            for c in range(_NC)]
    ys_ref[0] = jnp.dot(jnp.concatenate(ctxs, axis=0), w0_ref[...],
                        preferred_element_type=jnp.float32)


def _banded_attention(mc, xs, spf, Wq, Wk, Wv, W0):
    B, S, D = xs.shape
    grid = (B, S // (_NC * _TQ))
    return pl.pallas_call(
        _attn_body,
        grid=grid,
        in_specs=[
            pl.BlockSpec(memory_space=pltpu.SMEM),
            pl.BlockSpec((1, S, D), lambda b, t: (b, 0, 0)),
            pl.BlockSpec((1, S, 1), lambda b, t: (b, 0, 0)),
            pl.BlockSpec((D, D), lambda b, t: (0, 0)),
            pl.BlockSpec((D, D), lambda b, t: (0, 0)),
            pl.BlockSpec((D, D), lambda b, t: (0, 0)),
            pl.BlockSpec((D, D), lambda b, t: (0, 0)),
        ],
        out_specs=pl.BlockSpec((1, _NC * _TQ, D), lambda b, t: (b, t, 0)),
        out_shape=jax.ShapeDtypeStruct((B, S, D), jnp.float32),
    )(mc, xs, spf, Wq, Wk, Wv, W0)


def _sc_gather(table, idx):
    N, D = table.shape
    NW = 32
    CH = 128
    per_w = N // NW
    nch = per_w // CH
    idx3 = idx.reshape(NW, nch, CH)
    mesh = plsc.VectorSubcoreMesh(core_axis_name="c", subcore_axis_name="s")

    @functools.partial(
        pl.kernel, mesh=mesh,
        out_type=jax.ShapeDtypeStruct((N, D), jnp.float32),
        scratch_types=[
            pltpu.VMEM((nch, CH), jnp.int32),
            pltpu.VMEM((nch, CH, D), jnp.float32),
            pltpu.SemaphoreType.DMA,
        ],
    )
    def k(table_hbm, idx_hbm, out_hbm, idx_v, rows_v, sem):
        wid = lax.axis_index("s") * 2 + lax.axis_index("c")
        base = wid * per_w
        pltpu.sync_copy(idx_hbm.at[wid], idx_v)
        cps = [pltpu.async_copy(table_hbm.at[idx_v.at[j]], rows_v.at[j], sem)
               for j in range(nch)]
        for j in range(nch):
            cps[j].wait()
            pltpu.sync_copy(rows_v.at[j], out_hbm.at[pl.ds(base + j * CH, CH)])

    return k(table, idx3)


_TA = 256


def _maxgrp_body(pos_ref, mc_ref):
    S = pos_ref.shape[1]
    b = pl.program_id(0)
    t = pl.program_id(1)
    i0 = pl.multiple_of(t * _TA, _TA)
    a0 = pl.multiple_of(jnp.clip(i0 - 64, 0, S - _TW), 64)
    pqc = pos_ref[0, pl.ds(i0, _TA), :]
    pwc = pos_ref[0, pl.ds(a0, _TW), :]
    ii = lax.broadcasted_iota(jnp.int32, (_TW, _TW), 0)
    jj = lax.broadcasted_iota(jnp.int32, (_TW, _TW), 1)
    eye = (ii == jj).astype(jnp.float32)
    pw = lax.dot_general(pwc, eye, (((0,), (0,)), ((), ())),
                         preferred_element_type=jnp.float32)
    cnt = jnp.sum((pqc == pw).astype(jnp.int32), axis=1)
    mo = jnp.max(cnt)

    @pl.when((b == 0) & (t == 0))
    def _init():
        mc_ref[0, 0] = mo

    @pl.when(jnp.logical_not((b == 0) & (t == 0)))
    def _acc():
        mc_ref[0, 0] = jnp.maximum(mc_ref[0, 0], mo)


def _max_group(spf):
    B = spf.shape[0]
    S = spf.shape[1]
    return pl.pallas_call(
        _maxgrp_body,
        grid=(B, S // _TA),
        in_specs=[pl.BlockSpec((1, S, 1), lambda b, t: (b, 0, 0))],
        out_specs=pl.BlockSpec(memory_space=pltpu.SMEM),
        out_shape=jax.ShapeDtypeStruct((1, 1), jnp.int32),
    )(spf)


def kernel(embeddings, positions, Wq, Wk, Wv, W0):
    B, S, D = embeddings.shape
    perm = jnp.argsort(positions, axis=1).astype(jnp.int32)
    inv = jnp.argsort(perm, axis=1).astype(jnp.int32)
    sp = jnp.take_along_axis(positions, perm, axis=1)
    spf = sp.astype(jnp.float32).reshape(B, S, 1)

    row = (jnp.arange(B, dtype=jnp.int32) * S)[:, None]
    xs = _sc_gather(embeddings.reshape(B * S, D),
                    (perm + row).reshape(-1)).reshape(B, S, D)
    mc = _max_group(spf)
    ys = _banded_attention(mc, xs, spf, Wq * (1.0 / jnp.sqrt(_DH)),
                           Wk, Wv, W0)
    return _sc_gather(ys.reshape(B * S, D),
                      (inv + row).reshape(-1)).reshape(B, S, D)
